# Initial kernel scaffold; baseline (speedup 1.0000x reference)
import functools

import jax
import jax.numpy as jnp
from jax import lax
from jax.experimental import pallas as pl
from jax.experimental.pallas import tpu as pltpu
from jax.experimental.pallas import tpu_sc as plsc

_N = 10000
_E = 320000
_D = 128

_NC = 2
_NS = 16
_NW = _NC * _NS
_EPW = _E // _NW
_K = 80
_NBUF = 5
_NCHUNK = _EPW // _K
_NGROUP = _NCHUNK // _NBUF
_RPS = _N // _NS
_ZR = 125


def _seg_sum_partials(h, src, dst):
  mesh = plsc.VectorSubcoreMesh(core_axis_name="c", subcore_axis_name="s")

  @functools.partial(
      pl.kernel,
      out_type=jax.ShapeDtypeStruct((_NC, _N, _D), jnp.float32),
      mesh=mesh,
      scratch_types=[
          pltpu.VMEM((_NBUF, _K), jnp.int32),
          pltpu.VMEM((_NBUF, _K), jnp.int32),
          pltpu.VMEM((_NBUF, _K, _D), jnp.float32),
          pltpu.VMEM((_ZR, _D), jnp.float32),
          pltpu.VMEM_SHARED((_N, _D), jnp.float32),
          pltpu.SemaphoreType.DMA,
          pltpu.SemaphoreType.DMA,
      ],
  )
  def seg_kernel(h_hbm, src_hbm, dst_hbm, out_hbm,
                 srcb, dstb, rows, zbuf, acc, gsem, ssem):
    c = lax.axis_index("c")
    s = lax.axis_index("s")
    wid = s * _NC + c
    row0 = s * _RPS

    zvec = jnp.zeros((16,), jnp.float32)
    def _zrow(i, carry):
      for j in range(_D // 16):
        zbuf[i, pl.ds(j * 16, 16)] = zvec
      return carry
    lax.fori_loop(0, _ZR, _zrow, 0)
    for r in range(_RPS // _ZR):
      pltpu.sync_copy(zbuf, acc.at[pl.ds(row0 + r * _ZR, _ZR)])
    plsc.subcore_barrier()

    ebase = wid * _EPW

    def _group(g, carry):
      base = ebase + g * (_K * _NBUF)
      gh = []
      for b in range(_NBUF):
        pltpu.sync_copy(src_hbm.at[pl.ds(base + b * _K, _K)], srcb.at[b])
        pltpu.sync_copy(dst_hbm.at[pl.ds(base + b * _K, _K)], dstb.at[b])
        gh.append(pltpu.async_copy(h_hbm.at[srcb.at[b]], rows.at[b], gsem))
      sh = []
      for b in range(_NBUF):
        gh[b].wait()
        sh.append(pltpu.async_copy(rows.at[b], acc.at[dstb.at[b]], ssem,
                                   add=True))
      for b in range(_NBUF):
        sh[b].wait()
      return carry
    lax.fori_loop(0, _NGROUP, _group, 0)

    plsc.subcore_barrier()
    pltpu.sync_copy(acc.at[pl.ds(row0, _RPS)],
                    out_hbm.at[c, pl.ds(row0, _RPS)])

  return seg_kernel(h, src, dst)


_BR = 1000


def _ln_relu(x, g, b):
  def body(x_ref, g_ref, b_ref, o_ref):
    xx = x_ref[...]
    mu = jnp.mean(xx, axis=1, keepdims=True)
    var = jnp.mean((xx - mu) ** 2, axis=1, keepdims=True)
    xn = (xx - mu) * lax.rsqrt(var + 1e-5) * g_ref[...] + b_ref[...]
    o_ref[...] = jnp.maximum(xn, 0.0)

  return pl.pallas_call(
      body,
      grid=(_N // _BR,),
      in_specs=[
          pl.BlockSpec((_BR, _D), lambda i: (i, 0)),
          pl.BlockSpec((1, _D), lambda i: (0, 0)),
          pl.BlockSpec((1, _D), lambda i: (0, 0)),
      ],
      out_specs=pl.BlockSpec((_BR, _D), lambda i: (i, 0)),
      out_shape=jax.ShapeDtypeStruct((_N, _D), jnp.float32),
  )(x, g.reshape(1, _D), b.reshape(1, _D))


def _gconv_ln(p, h, WrT, br, WoT, g, b):
  def body(p_ref, h_ref, wr_ref, br_ref, wo_ref, g_ref, b_ref, o_ref):
    agg = p_ref[0] + p_ref[1]
    hh = h_ref[...]
    t = jnp.dot(agg, wr_ref[...], preferred_element_type=jnp.float32)
    t += jnp.dot(hh, wo_ref[...], preferred_element_type=jnp.float32)
    t = hh + jnp.maximum(t + br_ref[...], 0.0)
    mu = jnp.mean(t, axis=1, keepdims=True)
    var = jnp.mean((t - mu) ** 2, axis=1, keepdims=True)
    o_ref[...] = (t - mu) * lax.rsqrt(var + 1e-5) * g_ref[...] + b_ref[...]

  return pl.pallas_call(
      body,
      grid=(_N // _BR,),
      in_specs=[
          pl.BlockSpec((_NC, _BR, _D), lambda i: (0, i, 0)),
          pl.BlockSpec((_BR, _D), lambda i: (i, 0)),
          pl.BlockSpec((_D, _D), lambda i: (0, 0)),
          pl.BlockSpec((1, _D), lambda i: (0, 0)),
          pl.BlockSpec((_D, _D), lambda i: (0, 0)),
          pl.BlockSpec((1, _D), lambda i: (0, 0)),
          pl.BlockSpec((1, _D), lambda i: (0, 0)),
      ],
      out_specs=pl.BlockSpec((_BR, _D), lambda i: (i, 0)),
      out_shape=jax.ShapeDtypeStruct((_N, _D), jnp.float32),
  )(p, h, WrT, br.reshape(1, _D), WoT, g.reshape(1, _D), b.reshape(1, _D))


def _final_stage(p, h, WrT, br, WoT, g3, b3,
                 W1T, b1, gc, bc, W2T, b2, H2, NCLS):
  def body(p_ref, h_ref, wr_ref, br_ref, wo_ref, g3_ref, b3_ref,
           w1_ref, b1_ref, gc_ref, bc_ref, w2_ref, b2_ref,
           lg_ref, h3_ref):
    agg = p_ref[0] + p_ref[1]
    hh = h_ref[...]
    t = jnp.dot(agg, wr_ref[...], preferred_element_type=jnp.float32)
    t += jnp.dot(hh, wo_ref[...], preferred_element_type=jnp.float32)
    t = jnp.maximum(t + br_ref[...], 0.0)
    mu = jnp.mean(t, axis=1, keepdims=True)
    var = jnp.mean((t - mu) ** 2, axis=1, keepdims=True)
    h3 = (t - mu) * lax.rsqrt(var + 1e-5) * g3_ref[...] + b3_ref[...]
    h3_ref[...] = h3

    z = jnp.dot(h3, w1_ref[...], preferred_element_type=jnp.float32)
    z = jnp.maximum(z + b1_ref[...], 0.0)
    mu = jnp.mean(z, axis=1, keepdims=True)
    var = jnp.mean((z - mu) ** 2, axis=1, keepdims=True)
    z = (z - mu) * lax.rsqrt(var + 1e-5) * gc_ref[...] + bc_ref[...]
    lg = jnp.dot(z, w2_ref[...], preferred_element_type=jnp.float32)
    lg = lg + b2_ref[...]
    m = jnp.max(lg, axis=1, keepdims=True)
    e = jnp.exp(lg - m)
    lg_ref[...] = lg - m - jnp.log(jnp.sum(e, axis=1, keepdims=True))

  return pl.pallas_call(
      body,
      grid=(_N // _BR,),
      in_specs=[
          pl.BlockSpec((_NC, _BR, _D), lambda i: (0, i, 0)),
          pl.BlockSpec((_BR, _D), lambda i: (i, 0)),
          pl.BlockSpec((_D, _D), lambda i: (0, 0)),
          pl.BlockSpec((1, _D), lambda i: (0, 0)),
          pl.BlockSpec((_D, _D), lambda i: (0, 0)),
          pl.BlockSpec((1, _D), lambda i: (0, 0)),
          pl.BlockSpec((1, _D), lambda i: (0, 0)),
          pl.BlockSpec((_D, H2), lambda i: (0, 0)),
          pl.BlockSpec((1, H2), lambda i: (0, 0)),
          pl.BlockSpec((1, H2), lambda i: (0, 0)),
          pl.BlockSpec((1, H2), lambda i: (0, 0)),
          pl.BlockSpec((H2, NCLS), lambda i: (0, 0)),
          pl.BlockSpec((1, NCLS), lambda i: (0, 0)),
      ],
      out_specs=[
          pl.BlockSpec((_BR, NCLS), lambda i: (i, 0)),
          pl.BlockSpec((_BR, _D), lambda i: (i, 0)),
      ],
      out_shape=[
          jax.ShapeDtypeStruct((_N, NCLS), jnp.float32),
          jax.ShapeDtypeStruct((_N, _D), jnp.float32),
      ],
  )(p, h, WrT, br.reshape(1, _D), WoT, g3.reshape(1, _D), b3.reshape(1, _D),
    W1T, b1.reshape(1, H2), gc.reshape(1, H2), bc.reshape(1, H2),
    W2T, b2.reshape(1, NCLS))


def kernel(x, edge_index, ln0_g, ln0_b,
           W_rel1, b_rel1, W_root1, ln1_g, ln1_b,
           W_rel2, b_rel2, W_root2, ln2_g, ln2_b,
           W_rel3, b_rel3, W_root3, ln3_g, ln3_b,
           cls_W1, cls_b1, cls_ln_g, cls_ln_b, cls_W2, cls_b2):
  src = edge_index[0]
  dst = edge_index[1]
  H2 = cls_W1.shape[0]
  NCLS = cls_W2.shape[0]

  xn = _ln_relu(x, ln0_g, ln0_b)
  p1 = _seg_sum_partials(xn, src, dst)
  h1 = _gconv_ln(p1, xn, W_rel1.T, b_rel1, W_root1.T, ln1_g, ln1_b)
  p2 = _seg_sum_partials(h1, src, dst)
  h2 = _gconv_ln(p2, h1, W_rel2.T, b_rel2, W_root2.T, ln2_g, ln2_b)
  p3 = _seg_sum_partials(h2, src, dst)
  logits, h3 = _final_stage(
      p3, h2, W_rel3.T, b_rel3, W_root3.T, ln3_g, ln3_b,
      cls_W1.T, cls_b1, cls_ln_g, cls_ln_b, cls_W2.T, cls_b2, H2, NCLS)
  return (logits, h3)


# speedup vs baseline: 4.9747x; 4.9747x over previous
import functools

import jax
import jax.numpy as jnp
from jax import lax
from jax.experimental import pallas as pl
from jax.experimental.pallas import tpu as pltpu
from jax.experimental.pallas import tpu_sc as plsc

_N = 10000
_E = 320000
_D = 128

_NC = 2
_NS = 16
_NW = _NC * _NS
_EPW = _E // _NW
_K = 40
_NBUF = 5
_NCHUNK = _EPW // _K
_NGROUP = _NCHUNK // _NBUF
_RPS = 624
_TAIL = _N - _NS * _RPS


def _seg_sum_partials(h, src, dst):
  mesh = plsc.VectorSubcoreMesh(core_axis_name="c", subcore_axis_name="s")

  @functools.partial(
      pl.kernel,
      out_type=jax.ShapeDtypeStruct((_NC, _N, _D), jnp.float32),
      mesh=mesh,
      scratch_types=[
          pltpu.VMEM((_NBUF, _K), jnp.int32),
          pltpu.VMEM((_NBUF, _K), jnp.int32),
          pltpu.VMEM((_NBUF, _K, _D), jnp.float32),
          pltpu.VMEM_SHARED((_N, _D), jnp.float32),
          pltpu.SemaphoreType.DMA,
          pltpu.SemaphoreType.DMA,
      ],
  )
  def seg_kernel(h_hbm, src_hbm, dst_hbm, out_hbm,
                 srcb, dstb, rows, acc, gsem, ssem):
    c = lax.axis_index("c")
    s = lax.axis_index("s")
    wid = s * _NC + c
    row0 = s * _RPS

    zvec = jnp.zeros((16,), jnp.float32)
    def _zrow(i, carry):
      for b in range(_NBUF):
        for j in range(_D // 16):
          rows[b, i, pl.ds(j * 16, 16)] = zvec
      return carry
    lax.fori_loop(0, _K, _zrow, 0)
    nfull = _RPS // _K
    rem = _RPS - nfull * _K
    for r in range(nfull):
      pltpu.sync_copy(rows.at[r % _NBUF], acc.at[pl.ds(row0 + r * _K, _K)])
    pltpu.sync_copy(rows.at[0, pl.ds(0, rem)],
                    acc.at[pl.ds(row0 + nfull * _K, rem)])
    @pl.when(s == _NS - 1)
    def _zero_tail():
      pltpu.sync_copy(rows.at[1, pl.ds(0, _TAIL)],
                      acc.at[pl.ds(_NS * _RPS, _TAIL)])
    plsc.subcore_barrier()

    ebase = wid * _EPW

    def _group(g, carry):
      base = ebase + g * (_K * _NBUF)
      gh = []
      for b in range(_NBUF):
        pltpu.sync_copy(src_hbm.at[pl.ds(base + b * _K, _K)], srcb.at[b])
        pltpu.sync_copy(dst_hbm.at[pl.ds(base + b * _K, _K)], dstb.at[b])
        gh.append(pltpu.async_copy(h_hbm.at[srcb.at[b]], rows.at[b], gsem))
      sh = []
      for b in range(_NBUF):
        gh[b].wait()
        sh.append(pltpu.async_copy(rows.at[b], acc.at[dstb.at[b]], ssem,
                                   add=True))
      for b in range(_NBUF):
        sh[b].wait()
      return carry
    lax.fori_loop(0, _NGROUP, _group, 0)

    plsc.subcore_barrier()
    pltpu.sync_copy(acc.at[pl.ds(row0, _RPS)],
                    out_hbm.at[c, pl.ds(row0, _RPS)])
    @pl.when(s == _NS - 1)
    def _out_tail():
      pltpu.sync_copy(acc.at[pl.ds(_NS * _RPS, _TAIL)],
                      out_hbm.at[c, pl.ds(_NS * _RPS, _TAIL)])

  return seg_kernel(h, src, dst)


_BR = 1000


def _ln_relu(x, g, b):
  def body(x_ref, g_ref, b_ref, o_ref):
    xx = x_ref[...]
    mu = jnp.mean(xx, axis=1, keepdims=True)
    var = jnp.mean((xx - mu) ** 2, axis=1, keepdims=True)
    xn = (xx - mu) * lax.rsqrt(var + 1e-5) * g_ref[...] + b_ref[...]
    o_ref[...] = jnp.maximum(xn, 0.0)

  return pl.pallas_call(
      body,
      grid=(_N // _BR,),
      in_specs=[
          pl.BlockSpec((_BR, _D), lambda i: (i, 0)),
          pl.BlockSpec((1, _D), lambda i: (0, 0)),
          pl.BlockSpec((1, _D), lambda i: (0, 0)),
      ],
      out_specs=pl.BlockSpec((_BR, _D), lambda i: (i, 0)),
      out_shape=jax.ShapeDtypeStruct((_N, _D), jnp.float32),
  )(x, g.reshape(1, _D), b.reshape(1, _D))


def _gconv_ln(p, h, WrT, br, WoT, g, b):
  def body(p_ref, h_ref, wr_ref, br_ref, wo_ref, g_ref, b_ref, o_ref):
    agg = p_ref[0] + p_ref[1]
    hh = h_ref[...]
    t = jnp.dot(agg, wr_ref[...], preferred_element_type=jnp.float32)
    t += jnp.dot(hh, wo_ref[...], preferred_element_type=jnp.float32)
    t = hh + jnp.maximum(t + br_ref[...], 0.0)
    mu = jnp.mean(t, axis=1, keepdims=True)
    var = jnp.mean((t - mu) ** 2, axis=1, keepdims=True)
    o_ref[...] = (t - mu) * lax.rsqrt(var + 1e-5) * g_ref[...] + b_ref[...]

  return pl.pallas_call(
      body,
      grid=(_N // _BR,),
      in_specs=[
          pl.BlockSpec((_NC, _BR, _D), lambda i: (0, i, 0)),
          pl.BlockSpec((_BR, _D), lambda i: (i, 0)),
          pl.BlockSpec((_D, _D), lambda i: (0, 0)),
          pl.BlockSpec((1, _D), lambda i: (0, 0)),
          pl.BlockSpec((_D, _D), lambda i: (0, 0)),
          pl.BlockSpec((1, _D), lambda i: (0, 0)),
          pl.BlockSpec((1, _D), lambda i: (0, 0)),
      ],
      out_specs=pl.BlockSpec((_BR, _D), lambda i: (i, 0)),
      out_shape=jax.ShapeDtypeStruct((_N, _D), jnp.float32),
  )(p, h, WrT, br.reshape(1, _D), WoT, g.reshape(1, _D), b.reshape(1, _D))


def _final_stage(p, h, WrT, br, WoT, g3, b3,
                 W1T, b1, gc, bc, W2T, b2, H2, NCLS):
  def body(p_ref, h_ref, wr_ref, br_ref, wo_ref, g3_ref, b3_ref,
           w1_ref, b1_ref, gc_ref, bc_ref, w2_ref, b2_ref,
           lg_ref, h3_ref):
    agg = p_ref[0] + p_ref[1]
    hh = h_ref[...]
    t = jnp.dot(agg, wr_ref[...], preferred_element_type=jnp.float32)
    t += jnp.dot(hh, wo_ref[...], preferred_element_type=jnp.float32)
    t = jnp.maximum(t + br_ref[...], 0.0)
    mu = jnp.mean(t, axis=1, keepdims=True)
    var = jnp.mean((t - mu) ** 2, axis=1, keepdims=True)
    h3 = (t - mu) * lax.rsqrt(var + 1e-5) * g3_ref[...] + b3_ref[...]
    h3_ref[...] = h3

    z = jnp.dot(h3, w1_ref[...], preferred_element_type=jnp.float32)
    z = jnp.maximum(z + b1_ref[...], 0.0)
    mu = jnp.mean(z, axis=1, keepdims=True)
    var = jnp.mean((z - mu) ** 2, axis=1, keepdims=True)
    z = (z - mu) * lax.rsqrt(var + 1e-5) * gc_ref[...] + bc_ref[...]
    lg = jnp.dot(z, w2_ref[...], preferred_element_type=jnp.float32)
    lg = lg + b2_ref[...]
    m = jnp.max(lg, axis=1, keepdims=True)
    e = jnp.exp(lg - m)
    lg_ref[...] = lg - m - jnp.log(jnp.sum(e, axis=1, keepdims=True))

  return pl.pallas_call(
      body,
      grid=(_N // _BR,),
      in_specs=[
          pl.BlockSpec((_NC, _BR, _D), lambda i: (0, i, 0)),
          pl.BlockSpec((_BR, _D), lambda i: (i, 0)),
          pl.BlockSpec((_D, _D), lambda i: (0, 0)),
          pl.BlockSpec((1, _D), lambda i: (0, 0)),
          pl.BlockSpec((_D, _D), lambda i: (0, 0)),
          pl.BlockSpec((1, _D), lambda i: (0, 0)),
          pl.BlockSpec((1, _D), lambda i: (0, 0)),
          pl.BlockSpec((_D, H2), lambda i: (0, 0)),
          pl.BlockSpec((1, H2), lambda i: (0, 0)),
          pl.BlockSpec((1, H2), lambda i: (0, 0)),
          pl.BlockSpec((1, H2), lambda i: (0, 0)),
          pl.BlockSpec((H2, NCLS), lambda i: (0, 0)),
          pl.BlockSpec((1, NCLS), lambda i: (0, 0)),
      ],
      out_specs=[
          pl.BlockSpec((_BR, NCLS), lambda i: (i, 0)),
          pl.BlockSpec((_BR, _D), lambda i: (i, 0)),
      ],
      out_shape=[
          jax.ShapeDtypeStruct((_N, NCLS), jnp.float32),
          jax.ShapeDtypeStruct((_N, _D), jnp.float32),
      ],
  )(p, h, WrT, br.reshape(1, _D), WoT, g3.reshape(1, _D), b3.reshape(1, _D),
    W1T, b1.reshape(1, H2), gc.reshape(1, H2), bc.reshape(1, H2),
    W2T, b2.reshape(1, NCLS))


def kernel(x, edge_index, ln0_g, ln0_b,
           W_rel1, b_rel1, W_root1, ln1_g, ln1_b,
           W_rel2, b_rel2, W_root2, ln2_g, ln2_b,
           W_rel3, b_rel3, W_root3, ln3_g, ln3_b,
           cls_W1, cls_b1, cls_ln_g, cls_ln_b, cls_W2, cls_b2):
  src = edge_index[0]
  dst = edge_index[1]
  H2 = cls_W1.shape[0]
  NCLS = cls_W2.shape[0]

  xn = _ln_relu(x, ln0_g, ln0_b)
  p1 = _seg_sum_partials(xn, src, dst)
  h1 = _gconv_ln(p1, xn, W_rel1.T, b_rel1, W_root1.T, ln1_g, ln1_b)
  p2 = _seg_sum_partials(h1, src, dst)
  h2 = _gconv_ln(p2, h1, W_rel2.T, b_rel2, W_root2.T, ln2_g, ln2_b)
  p3 = _seg_sum_partials(h2, src, dst)
  logits, h3 = _final_stage(
      p3, h2, W_rel3.T, b_rel3, W_root3.T, ln3_g, ln3_b,
      cls_W1.T, cls_b1, cls_ln_g, cls_ln_b, cls_W2.T, cls_b2, H2, NCLS)
  return (logits, h3)


# speedup vs baseline: 10.6911x; 2.1491x over previous
import functools

import jax
import jax.numpy as jnp
from jax import lax
from jax.experimental import pallas as pl
from jax.experimental.pallas import tpu as pltpu
from jax.experimental.pallas import tpu_sc as plsc

_N = 10000
_E = 320000
_D = 128

_NC = 2
_NS = 16
_NW = _NC * _NS
_EPW = _E // _NW
_K = 40
_NBUF = 5
_NCHUNK = _EPW // _K
_NGROUP = _NCHUNK // _NBUF
_RPS = 624
_TAIL = _N - _NS * _RPS


def _seg_sum_partials(h, src3, dst3):
  mesh = plsc.VectorSubcoreMesh(core_axis_name="c", subcore_axis_name="s")

  @functools.partial(
      pl.kernel,
      out_type=jax.ShapeDtypeStruct((_NC, _N, _D), jnp.float32),
      mesh=mesh,
      scratch_types=[
          pltpu.VMEM((_EPW,), jnp.int32),
          pltpu.VMEM((_NBUF, _K), jnp.int32),
          pltpu.VMEM((_NBUF, _K, _D), jnp.float32),
          pltpu.VMEM_SHARED((_N, _D), jnp.float32),
          pltpu.SemaphoreType.DMA,
          pltpu.SemaphoreType.DMA,
          pltpu.SemaphoreType.DMA,
      ],
  )
  def seg_kernel(h_hbm, src_hbm, dst_hbm, out_hbm,
                 srcb, dstb, rows, acc, gsem, ssem, isem):
    c = lax.axis_index("c")
    s = lax.axis_index("s")
    wid = s * _NC + c
    row0 = s * _RPS

    pltpu.sync_copy(src_hbm.at[wid], srcb)

    zvec = jnp.zeros((16,), jnp.float32)
    def _zrow(i, carry):
      for b in range(_NBUF):
        for j in range(_D // 16):
          rows[b, i, pl.ds(j * 16, 16)] = zvec
      return carry
    lax.fori_loop(0, _K, _zrow, 0)
    nfull = _RPS // _K
    rem = _RPS - nfull * _K
    for r in range(nfull):
      pltpu.sync_copy(rows.at[r % _NBUF], acc.at[pl.ds(row0 + r * _K, _K)])
    pltpu.sync_copy(rows.at[0, pl.ds(0, rem)],
                    acc.at[pl.ds(row0 + nfull * _K, rem)])
    @pl.when(s == _NS - 1)
    def _zero_tail():
      pltpu.sync_copy(rows.at[1, pl.ds(0, _TAIL)],
                      acc.at[pl.ds(_NS * _RPS, _TAIL)])
    plsc.subcore_barrier()

    def _fire(chunk, b):
      pltpu.async_copy(dst_hbm.at[wid, chunk], dstb.at[b], isem)
      pltpu.async_copy(h_hbm.at[srcb.at[pl.ds(chunk * _K, _K)]],
                       rows.at[b], gsem)

    def _drain(b):
      pltpu.make_async_copy(dst_hbm.at[0, 0], dstb.at[b], isem).wait()
      pltpu.make_async_copy(h_hbm.at[pl.ds(0, _K)], rows.at[b], gsem).wait()

    for b in range(_NBUF):
      _fire(b, b)

    def _group(g, carry):
      sh = []
      for b in range(_NBUF):
        _drain(b)
        sh.append(pltpu.async_copy(rows.at[b], acc.at[dstb.at[b]],
                                   ssem, add=True))
      for b in range(_NBUF):
        sh[b].wait()
        _fire((g + 1) * _NBUF + b, b)
      return carry
    lax.fori_loop(0, _NGROUP - 1, _group, 0)

    sh = []
    for b in range(_NBUF):
      _drain(b)
      sh.append(pltpu.async_copy(rows.at[b], acc.at[dstb.at[b]],
                                 ssem, add=True))
    for b in range(_NBUF):
      sh[b].wait()

    plsc.subcore_barrier()
    pltpu.sync_copy(acc.at[pl.ds(row0, _RPS)],
                    out_hbm.at[c, pl.ds(row0, _RPS)])
    @pl.when(s == _NS - 1)
    def _out_tail():
      pltpu.sync_copy(acc.at[pl.ds(_NS * _RPS, _TAIL)],
                      out_hbm.at[c, pl.ds(_NS * _RPS, _TAIL)])

  return seg_kernel(h, src3, dst3)


_BR = 1000


def _ln_relu(x, g, b):
  def body(x_ref, g_ref, b_ref, o_ref):
    xx = x_ref[...]
    mu = jnp.mean(xx, axis=1, keepdims=True)
    var = jnp.mean((xx - mu) ** 2, axis=1, keepdims=True)
    xn = (xx - mu) * lax.rsqrt(var + 1e-5) * g_ref[...] + b_ref[...]
    o_ref[...] = jnp.maximum(xn, 0.0)

  return pl.pallas_call(
      body,
      grid=(_N // _BR,),
      in_specs=[
          pl.BlockSpec((_BR, _D), lambda i: (i, 0)),
          pl.BlockSpec((1, _D), lambda i: (0, 0)),
          pl.BlockSpec((1, _D), lambda i: (0, 0)),
      ],
      out_specs=pl.BlockSpec((_BR, _D), lambda i: (i, 0)),
      out_shape=jax.ShapeDtypeStruct((_N, _D), jnp.float32),
  )(x, g.reshape(1, _D), b.reshape(1, _D))


def _gconv_ln(p, h, WrT, br, WoT, g, b):
  def body(p_ref, h_ref, wr_ref, br_ref, wo_ref, g_ref, b_ref, o_ref):
    agg = p_ref[0] + p_ref[1]
    hh = h_ref[...]
    t = jnp.dot(agg, wr_ref[...], preferred_element_type=jnp.float32)
    t += jnp.dot(hh, wo_ref[...], preferred_element_type=jnp.float32)
    t = hh + jnp.maximum(t + br_ref[...], 0.0)
    mu = jnp.mean(t, axis=1, keepdims=True)
    var = jnp.mean((t - mu) ** 2, axis=1, keepdims=True)
    o_ref[...] = (t - mu) * lax.rsqrt(var + 1e-5) * g_ref[...] + b_ref[...]

  return pl.pallas_call(
      body,
      grid=(_N // _BR,),
      in_specs=[
          pl.BlockSpec((_NC, _BR, _D), lambda i: (0, i, 0)),
          pl.BlockSpec((_BR, _D), lambda i: (i, 0)),
          pl.BlockSpec((_D, _D), lambda i: (0, 0)),
          pl.BlockSpec((1, _D), lambda i: (0, 0)),
          pl.BlockSpec((_D, _D), lambda i: (0, 0)),
          pl.BlockSpec((1, _D), lambda i: (0, 0)),
          pl.BlockSpec((1, _D), lambda i: (0, 0)),
      ],
      out_specs=pl.BlockSpec((_BR, _D), lambda i: (i, 0)),
      out_shape=jax.ShapeDtypeStruct((_N, _D), jnp.float32),
  )(p, h, WrT, br.reshape(1, _D), WoT, g.reshape(1, _D), b.reshape(1, _D))


def _final_stage(p, h, WrT, br, WoT, g3, b3,
                 W1T, b1, gc, bc, W2T, b2, H2, NCLS):
  def body(p_ref, h_ref, wr_ref, br_ref, wo_ref, g3_ref, b3_ref,
           w1_ref, b1_ref, gc_ref, bc_ref, w2_ref, b2_ref,
           lg_ref, h3_ref):
    agg = p_ref[0] + p_ref[1]
    hh = h_ref[...]
    t = jnp.dot(agg, wr_ref[...], preferred_element_type=jnp.float32)
    t += jnp.dot(hh, wo_ref[...], preferred_element_type=jnp.float32)
    t = jnp.maximum(t + br_ref[...], 0.0)
    mu = jnp.mean(t, axis=1, keepdims=True)
    var = jnp.mean((t - mu) ** 2, axis=1, keepdims=True)
    h3 = (t - mu) * lax.rsqrt(var + 1e-5) * g3_ref[...] + b3_ref[...]
    h3_ref[...] = h3

    z = jnp.dot(h3, w1_ref[...], preferred_element_type=jnp.float32)
    z = jnp.maximum(z + b1_ref[...], 0.0)
    mu = jnp.mean(z, axis=1, keepdims=True)
    var = jnp.mean((z - mu) ** 2, axis=1, keepdims=True)
    z = (z - mu) * lax.rsqrt(var + 1e-5) * gc_ref[...] + bc_ref[...]
    lg = jnp.dot(z, w2_ref[...], preferred_element_type=jnp.float32)
    lg = lg + b2_ref[...]
    m = jnp.max(lg, axis=1, keepdims=True)
    e = jnp.exp(lg - m)
    lg_ref[...] = lg - m - jnp.log(jnp.sum(e, axis=1, keepdims=True))

  return pl.pallas_call(
      body,
      grid=(_N // _BR,),
      in_specs=[
          pl.BlockSpec((_NC, _BR, _D), lambda i: (0, i, 0)),
          pl.BlockSpec((_BR, _D), lambda i: (i, 0)),
          pl.BlockSpec((_D, _D), lambda i: (0, 0)),
          pl.BlockSpec((1, _D), lambda i: (0, 0)),
          pl.BlockSpec((_D, _D), lambda i: (0, 0)),
          pl.BlockSpec((1, _D), lambda i: (0, 0)),
          pl.BlockSpec((1, _D), lambda i: (0, 0)),
          pl.BlockSpec((_D, H2), lambda i: (0, 0)),
          pl.BlockSpec((1, H2), lambda i: (0, 0)),
          pl.BlockSpec((1, H2), lambda i: (0, 0)),
          pl.BlockSpec((1, H2), lambda i: (0, 0)),
          pl.BlockSpec((H2, NCLS), lambda i: (0, 0)),
          pl.BlockSpec((1, NCLS), lambda i: (0, 0)),
      ],
      out_specs=[
          pl.BlockSpec((_BR, NCLS), lambda i: (i, 0)),
          pl.BlockSpec((_BR, _D), lambda i: (i, 0)),
      ],
      out_shape=[
          jax.ShapeDtypeStruct((_N, NCLS), jnp.float32),
          jax.ShapeDtypeStruct((_N, _D), jnp.float32),
      ],
  )(p, h, WrT, br.reshape(1, _D), WoT, g3.reshape(1, _D), b3.reshape(1, _D),
    W1T, b1.reshape(1, H2), gc.reshape(1, H2), bc.reshape(1, H2),
    W2T, b2.reshape(1, NCLS))


def kernel(x, edge_index, ln0_g, ln0_b,
           W_rel1, b_rel1, W_root1, ln1_g, ln1_b,
           W_rel2, b_rel2, W_root2, ln2_g, ln2_b,
           W_rel3, b_rel3, W_root3, ln3_g, ln3_b,
           cls_W1, cls_b1, cls_ln_g, cls_ln_b, cls_W2, cls_b2):
  src = edge_index[0].reshape(_NW, _EPW)
  dst = edge_index[1].reshape(_NW, _NCHUNK, _K)
  H2 = cls_W1.shape[0]
  NCLS = cls_W2.shape[0]

  xn = _ln_relu(x, ln0_g, ln0_b)
  p1 = _seg_sum_partials(xn, src, dst)
  h1 = _gconv_ln(p1, xn, W_rel1.T, b_rel1, W_root1.T, ln1_g, ln1_b)
  p2 = _seg_sum_partials(h1, src, dst)
  h2 = _gconv_ln(p2, h1, W_rel2.T, b_rel2, W_root2.T, ln2_g, ln2_b)
  p3 = _seg_sum_partials(h2, src, dst)
  logits, h3 = _final_stage(
      p3, h2, W_rel3.T, b_rel3, W_root3.T, ln3_g, ln3_b,
      cls_W1.T, cls_b1, cls_ln_g, cls_ln_b, cls_W2.T, cls_b2, H2, NCLS)
  return (logits, h3)


# speedup vs baseline: 10.8017x; 1.0103x over previous
import functools

import jax
import jax.numpy as jnp
from jax import lax
from jax.experimental import pallas as pl
from jax.experimental.pallas import tpu as pltpu
from jax.experimental.pallas import tpu_sc as plsc

_N = 10000
_E = 320000
_D = 128

_NC = 2
_NS = 16
_NW = _NC * _NS
_EPW = _E // _NW
_K = 40
_NBUF = 5
_NCHUNK = _EPW // _K
_NGROUP = _NCHUNK // _NBUF
_RPS = 624
_TAIL = _N - _NS * _RPS


def _seg_sum_partials(h, src3, dst3):
  mesh = plsc.VectorSubcoreMesh(core_axis_name="c", subcore_axis_name="s")

  @functools.partial(
      pl.kernel,
      out_type=jax.ShapeDtypeStruct((_NC, _N, _D), jnp.float32),
      mesh=mesh,
      scratch_types=[
          pltpu.VMEM((_EPW,), jnp.int32),
          pltpu.VMEM((_NBUF, _K), jnp.int32),
          pltpu.VMEM((_NBUF, _K, _D), jnp.float32),
          pltpu.VMEM_SHARED((_N, _D), jnp.float32),
          pltpu.SemaphoreType.DMA,
          pltpu.SemaphoreType.DMA,
          pltpu.SemaphoreType.DMA,
      ],
  )
  def seg_kernel(h_hbm, src_hbm, dst_hbm, out_hbm,
                 srcb, dstb, rows, acc, gsem, ssem, isem):
    c = lax.axis_index("c")
    s = lax.axis_index("s")
    wid = s * _NC + c
    row0 = s * _RPS

    pltpu.sync_copy(src_hbm.at[wid], srcb)

    zvec = jnp.zeros((16,), jnp.float32)
    def _zrow(i, carry):
      for b in range(_NBUF):
        for j in range(_D // 16):
          rows[b, i, pl.ds(j * 16, 16)] = zvec
      return carry
    lax.fori_loop(0, _K, _zrow, 0)
    nfull = _RPS // _K
    rem = _RPS - nfull * _K
    for r in range(nfull):
      pltpu.sync_copy(rows.at[r % _NBUF], acc.at[pl.ds(row0 + r * _K, _K)])
    pltpu.sync_copy(rows.at[0, pl.ds(0, rem)],
                    acc.at[pl.ds(row0 + nfull * _K, rem)])
    @pl.when(s == _NS - 1)
    def _zero_tail():
      pltpu.sync_copy(rows.at[1, pl.ds(0, _TAIL)],
                      acc.at[pl.ds(_NS * _RPS, _TAIL)])
    plsc.subcore_barrier()

    def _fire(chunk, b):
      pltpu.async_copy(dst_hbm.at[wid, chunk], dstb.at[b], isem)
      pltpu.async_copy(h_hbm.at[srcb.at[pl.ds(chunk * _K, _K)]],
                       rows.at[b], gsem)

    def _drain(b):
      pltpu.make_async_copy(dst_hbm.at[0, 0], dstb.at[b], isem).wait()
      pltpu.make_async_copy(h_hbm.at[pl.ds(0, _K)], rows.at[b], gsem).wait()

    for b in range(_NBUF):
      _fire(b, b)

    def _group(g, carry):
      sh = []
      for b in range(_NBUF):
        _drain(b)
        sh.append(pltpu.async_copy(rows.at[b], acc.at[dstb.at[b]],
                                   ssem, add=True))
      for b in range(_NBUF):
        sh[b].wait()
        _fire((g + 1) * _NBUF + b, b)
      return carry
    lax.fori_loop(0, _NGROUP - 1, _group, 0)

    sh = []
    for b in range(_NBUF):
      _drain(b)
      sh.append(pltpu.async_copy(rows.at[b], acc.at[dstb.at[b]],
                                 ssem, add=True))
    for b in range(_NBUF):
      sh[b].wait()

    plsc.subcore_barrier()
    pltpu.sync_copy(acc.at[pl.ds(row0, _RPS)],
                    out_hbm.at[c, pl.ds(row0, _RPS)])
    @pl.when(s == _NS - 1)
    def _out_tail():
      pltpu.sync_copy(acc.at[pl.ds(_NS * _RPS, _TAIL)],
                      out_hbm.at[c, pl.ds(_NS * _RPS, _TAIL)])

  return seg_kernel(h, src3, dst3)


_BR = 1000


def _ln_relu(x, g, b):
  def body(x_ref, g_ref, b_ref, o_ref):
    xx = x_ref[...]
    mu = jnp.mean(xx, axis=1, keepdims=True)
    var = jnp.mean((xx - mu) ** 2, axis=1, keepdims=True)
    xn = (xx - mu) * lax.rsqrt(var + 1e-5) * g_ref[...] + b_ref[...]
    o_ref[...] = jnp.maximum(xn, 0.0)

  return pl.pallas_call(
      body,
      grid=(_N // _BR,),
      in_specs=[
          pl.BlockSpec((_BR, _D), lambda i: (i, 0)),
          pl.BlockSpec((1, _D), lambda i: (0, 0)),
          pl.BlockSpec((1, _D), lambda i: (0, 0)),
      ],
      out_specs=pl.BlockSpec((_BR, _D), lambda i: (i, 0)),
      out_shape=jax.ShapeDtypeStruct((_N, _D), jnp.float32),
  )(x, g.reshape(1, _D), b.reshape(1, _D))


def _gconv_ln(p, h, WrT, br, WoT, g, b):
  def body(p_ref, h_ref, wr_ref, br_ref, wo_ref, g_ref, b_ref, o_ref):
    agg = p_ref[0] + p_ref[1]
    hh = h_ref[...]
    t = jnp.dot(agg.astype(jnp.bfloat16), wr_ref[...].astype(jnp.bfloat16),
                preferred_element_type=jnp.float32)
    t += jnp.dot(hh.astype(jnp.bfloat16), wo_ref[...].astype(jnp.bfloat16),
                 preferred_element_type=jnp.float32)
    t = hh + jnp.maximum(t + br_ref[...], 0.0)
    mu = jnp.mean(t, axis=1, keepdims=True)
    var = jnp.mean((t - mu) ** 2, axis=1, keepdims=True)
    o_ref[...] = (t - mu) * lax.rsqrt(var + 1e-5) * g_ref[...] + b_ref[...]

  return pl.pallas_call(
      body,
      grid=(_N // _BR,),
      in_specs=[
          pl.BlockSpec((_NC, _BR, _D), lambda i: (0, i, 0)),
          pl.BlockSpec((_BR, _D), lambda i: (i, 0)),
          pl.BlockSpec((_D, _D), lambda i: (0, 0)),
          pl.BlockSpec((1, _D), lambda i: (0, 0)),
          pl.BlockSpec((_D, _D), lambda i: (0, 0)),
          pl.BlockSpec((1, _D), lambda i: (0, 0)),
          pl.BlockSpec((1, _D), lambda i: (0, 0)),
      ],
      out_specs=pl.BlockSpec((_BR, _D), lambda i: (i, 0)),
      out_shape=jax.ShapeDtypeStruct((_N, _D), jnp.float32),
  )(p, h, WrT, br.reshape(1, _D), WoT, g.reshape(1, _D), b.reshape(1, _D))


def _final_stage(p, h, WrT, br, WoT, g3, b3,
                 W1T, b1, gc, bc, W2T, b2, H2, NCLS):
  def body(p_ref, h_ref, wr_ref, br_ref, wo_ref, g3_ref, b3_ref,
           w1_ref, b1_ref, gc_ref, bc_ref, w2_ref, b2_ref,
           lg_ref, h3_ref):
    agg = p_ref[0] + p_ref[1]
    hh = h_ref[...]
    t = jnp.dot(agg.astype(jnp.bfloat16), wr_ref[...].astype(jnp.bfloat16),
                preferred_element_type=jnp.float32)
    t += jnp.dot(hh.astype(jnp.bfloat16), wo_ref[...].astype(jnp.bfloat16),
                 preferred_element_type=jnp.float32)
    t = jnp.maximum(t + br_ref[...], 0.0)
    mu = jnp.mean(t, axis=1, keepdims=True)
    var = jnp.mean((t - mu) ** 2, axis=1, keepdims=True)
    h3 = (t - mu) * lax.rsqrt(var + 1e-5) * g3_ref[...] + b3_ref[...]
    h3_ref[...] = h3

    z = jnp.dot(h3.astype(jnp.bfloat16), w1_ref[...].astype(jnp.bfloat16),
                preferred_element_type=jnp.float32)
    z = jnp.maximum(z + b1_ref[...], 0.0)
    mu = jnp.mean(z, axis=1, keepdims=True)
    var = jnp.mean((z - mu) ** 2, axis=1, keepdims=True)
    z = (z - mu) * lax.rsqrt(var + 1e-5) * gc_ref[...] + bc_ref[...]
    lg = jnp.dot(z.astype(jnp.bfloat16), w2_ref[...].astype(jnp.bfloat16),
                 preferred_element_type=jnp.float32)
    lg = lg + b2_ref[...]
    m = jnp.max(lg, axis=1, keepdims=True)
    e = jnp.exp(lg - m)
    lg_ref[...] = lg - m - jnp.log(jnp.sum(e, axis=1, keepdims=True))

  return pl.pallas_call(
      body,
      grid=(_N // _BR,),
      in_specs=[
          pl.BlockSpec((_NC, _BR, _D), lambda i: (0, i, 0)),
          pl.BlockSpec((_BR, _D), lambda i: (i, 0)),
          pl.BlockSpec((_D, _D), lambda i: (0, 0)),
          pl.BlockSpec((1, _D), lambda i: (0, 0)),
          pl.BlockSpec((_D, _D), lambda i: (0, 0)),
          pl.BlockSpec((1, _D), lambda i: (0, 0)),
          pl.BlockSpec((1, _D), lambda i: (0, 0)),
          pl.BlockSpec((_D, H2), lambda i: (0, 0)),
          pl.BlockSpec((1, H2), lambda i: (0, 0)),
          pl.BlockSpec((1, H2), lambda i: (0, 0)),
          pl.BlockSpec((1, H2), lambda i: (0, 0)),
          pl.BlockSpec((H2, NCLS), lambda i: (0, 0)),
          pl.BlockSpec((1, NCLS), lambda i: (0, 0)),
      ],
      out_specs=[
          pl.BlockSpec((_BR, NCLS), lambda i: (i, 0)),
          pl.BlockSpec((_BR, _D), lambda i: (i, 0)),
      ],
      out_shape=[
          jax.ShapeDtypeStruct((_N, NCLS), jnp.float32),
          jax.ShapeDtypeStruct((_N, _D), jnp.float32),
      ],
  )(p, h, WrT, br.reshape(1, _D), WoT, g3.reshape(1, _D), b3.reshape(1, _D),
    W1T, b1.reshape(1, H2), gc.reshape(1, H2), bc.reshape(1, H2),
    W2T, b2.reshape(1, NCLS))


def kernel(x, edge_index, ln0_g, ln0_b,
           W_rel1, b_rel1, W_root1, ln1_g, ln1_b,
           W_rel2, b_rel2, W_root2, ln2_g, ln2_b,
           W_rel3, b_rel3, W_root3, ln3_g, ln3_b,
           cls_W1, cls_b1, cls_ln_g, cls_ln_b, cls_W2, cls_b2):
  src = edge_index[0].reshape(_NW, _EPW)
  dst = edge_index[1].reshape(_NW, _NCHUNK, _K)
  H2 = cls_W1.shape[0]
  NCLS = cls_W2.shape[0]

  xn = _ln_relu(x, ln0_g, ln0_b)
  p1 = _seg_sum_partials(xn, src, dst)
  h1 = _gconv_ln(p1, xn, W_rel1.T, b_rel1, W_root1.T, ln1_g, ln1_b)
  p2 = _seg_sum_partials(h1, src, dst)
  h2 = _gconv_ln(p2, h1, W_rel2.T, b_rel2, W_root2.T, ln2_g, ln2_b)
  p3 = _seg_sum_partials(h2, src, dst)
  logits, h3 = _final_stage(
      p3, h2, W_rel3.T, b_rel3, W_root3.T, ln3_g, ln3_b,
      cls_W1.T, cls_b1, cls_ln_g, cls_ln_b, cls_W2.T, cls_b2, H2, NCLS)
  return (logits, h3)


# speedup vs baseline: 10.9631x; 1.0149x over previous
import functools

import jax
import jax.numpy as jnp
from jax import lax
from jax.experimental import pallas as pl
from jax.experimental.pallas import tpu as pltpu
from jax.experimental.pallas import tpu_sc as plsc

_N = 10000
_E = 320000
_D = 128

_NC = 2
_NS = 16
_NW = _NC * _NS
_EPW = _E // _NW
_K = 40
_NBUF = 5
_NCHUNK = _EPW // _K
_NGROUP = _NCHUNK // _NBUF
_RPS = 624
_TAIL = _N - _NS * _RPS


def _seg_sum_partials(h, src, dst):
  mesh = plsc.VectorSubcoreMesh(core_axis_name="c", subcore_axis_name="s")

  @functools.partial(
      pl.kernel,
      out_type=jax.ShapeDtypeStruct((_NC, _N, _D), jnp.float32),
      mesh=mesh,
      scratch_types=[
          pltpu.VMEM((_EPW,), jnp.int32),
          pltpu.VMEM((_NBUF, _K), jnp.int32),
          pltpu.VMEM((_NBUF, _K, _D), jnp.float32),
          pltpu.VMEM_SHARED((_N, _D), jnp.float32),
          pltpu.SemaphoreType.DMA,
          pltpu.SemaphoreType.DMA,
          pltpu.SemaphoreType.DMA,
      ],
  )
  def seg_kernel(h_hbm, src_hbm, dst_hbm, out_hbm,
                 srcb, dstb, rows, acc, gsem, ssem, isem):
    c = lax.axis_index("c")
    s = lax.axis_index("s")
    wid = s * _NC + c
    row0 = s * _RPS
    ebase = wid * _EPW

    pltpu.sync_copy(src_hbm.at[pl.ds(ebase, _EPW)], srcb)

    zvec = jnp.zeros((16,), jnp.float32)
    def _zrow(i, carry):
      for b in range(_NBUF):
        for j in range(_D // 16):
          rows[b, i, pl.ds(j * 16, 16)] = zvec
      return carry
    lax.fori_loop(0, _K, _zrow, 0)
    nfull = _RPS // _K
    rem = _RPS - nfull * _K
    for r in range(nfull):
      pltpu.sync_copy(rows.at[r % _NBUF], acc.at[pl.ds(row0 + r * _K, _K)])
    pltpu.sync_copy(rows.at[0, pl.ds(0, rem)],
                    acc.at[pl.ds(row0 + nfull * _K, rem)])
    @pl.when(s == _NS - 1)
    def _zero_tail():
      pltpu.sync_copy(rows.at[1, pl.ds(0, _TAIL)],
                      acc.at[pl.ds(_NS * _RPS, _TAIL)])
    plsc.subcore_barrier()

    def _fire(chunk, b):
      pltpu.async_copy(dst_hbm.at[pl.ds(ebase + chunk * _K, _K)],
                       dstb.at[b], isem)
      pltpu.async_copy(h_hbm.at[srcb.at[pl.ds(chunk * _K, _K)]],
                       rows.at[b], gsem)

    def _drain(b):
      pltpu.make_async_copy(dst_hbm.at[pl.ds(0, _K)], dstb.at[b],
                            isem).wait()
      pltpu.make_async_copy(h_hbm.at[pl.ds(0, _K)], rows.at[b], gsem).wait()

    for b in range(_NBUF):
      _fire(b, b)

    def _group(g, carry):
      sh = []
      for b in range(_NBUF):
        _drain(b)
        sh.append(pltpu.async_copy(rows.at[b], acc.at[dstb.at[b]],
                                   ssem, add=True))
      for b in range(_NBUF):
        sh[b].wait()
        _fire((g + 1) * _NBUF + b, b)
      return carry
    lax.fori_loop(0, _NGROUP - 1, _group, 0)

    sh = []
    for b in range(_NBUF):
      _drain(b)
      sh.append(pltpu.async_copy(rows.at[b], acc.at[dstb.at[b]],
                                 ssem, add=True))
    for b in range(_NBUF):
      sh[b].wait()

    plsc.subcore_barrier()
    pltpu.sync_copy(acc.at[pl.ds(row0, _RPS)],
                    out_hbm.at[c, pl.ds(row0, _RPS)])
    @pl.when(s == _NS - 1)
    def _out_tail():
      pltpu.sync_copy(acc.at[pl.ds(_NS * _RPS, _TAIL)],
                      out_hbm.at[c, pl.ds(_NS * _RPS, _TAIL)])

  return seg_kernel(h, src, dst)


_BR = 1000


def _ln_relu(x, g, b):
  def body(x_ref, g_ref, b_ref, o_ref):
    xx = x_ref[...]
    mu = jnp.mean(xx, axis=1, keepdims=True)
    var = jnp.mean((xx - mu) ** 2, axis=1, keepdims=True)
    xn = (xx - mu) * lax.rsqrt(var + 1e-5) * g_ref[...] + b_ref[...]
    o_ref[...] = jnp.maximum(xn, 0.0)

  return pl.pallas_call(
      body,
      grid=(_N // _BR,),
      in_specs=[
          pl.BlockSpec((_BR, _D), lambda i: (i, 0)),
          pl.BlockSpec((1, _D), lambda i: (0, 0)),
          pl.BlockSpec((1, _D), lambda i: (0, 0)),
      ],
      out_specs=pl.BlockSpec((_BR, _D), lambda i: (i, 0)),
      out_shape=jax.ShapeDtypeStruct((_N, _D), jnp.float32),
  )(x, g.reshape(1, _D), b.reshape(1, _D))


def _gconv_ln(p, h, WrT, br, WoT, g, b):
  def body(p_ref, h_ref, wr_ref, br_ref, wo_ref, g_ref, b_ref, o_ref):
    agg = p_ref[0] + p_ref[1]
    hh = h_ref[...]
    t = jnp.dot(agg.astype(jnp.bfloat16), wr_ref[...].astype(jnp.bfloat16),
                preferred_element_type=jnp.float32)
    t += jnp.dot(hh.astype(jnp.bfloat16), wo_ref[...].astype(jnp.bfloat16),
                 preferred_element_type=jnp.float32)
    t = hh + jnp.maximum(t + br_ref[...], 0.0)
    mu = jnp.mean(t, axis=1, keepdims=True)
    var = jnp.mean((t - mu) ** 2, axis=1, keepdims=True)
    o_ref[...] = (t - mu) * lax.rsqrt(var + 1e-5) * g_ref[...] + b_ref[...]

  return pl.pallas_call(
      body,
      grid=(_N // _BR,),
      in_specs=[
          pl.BlockSpec((_NC, _BR, _D), lambda i: (0, i, 0)),
          pl.BlockSpec((_BR, _D), lambda i: (i, 0)),
          pl.BlockSpec((_D, _D), lambda i: (0, 0)),
          pl.BlockSpec((1, _D), lambda i: (0, 0)),
          pl.BlockSpec((_D, _D), lambda i: (0, 0)),
          pl.BlockSpec((1, _D), lambda i: (0, 0)),
          pl.BlockSpec((1, _D), lambda i: (0, 0)),
      ],
      out_specs=pl.BlockSpec((_BR, _D), lambda i: (i, 0)),
      out_shape=jax.ShapeDtypeStruct((_N, _D), jnp.float32),
  )(p, h, WrT, br.reshape(1, _D), WoT, g.reshape(1, _D), b.reshape(1, _D))


def _final_stage(p, h, WrT, br, WoT, g3, b3,
                 W1T, b1, gc, bc, W2T, b2, H2, NCLS):
  def body(p_ref, h_ref, wr_ref, br_ref, wo_ref, g3_ref, b3_ref,
           w1_ref, b1_ref, gc_ref, bc_ref, w2_ref, b2_ref,
           lg_ref, h3_ref):
    agg = p_ref[0] + p_ref[1]
    hh = h_ref[...]
    t = jnp.dot(agg.astype(jnp.bfloat16), wr_ref[...].astype(jnp.bfloat16),
                preferred_element_type=jnp.float32)
    t += jnp.dot(hh.astype(jnp.bfloat16), wo_ref[...].astype(jnp.bfloat16),
                 preferred_element_type=jnp.float32)
    t = jnp.maximum(t + br_ref[...], 0.0)
    mu = jnp.mean(t, axis=1, keepdims=True)
    var = jnp.mean((t - mu) ** 2, axis=1, keepdims=True)
    h3 = (t - mu) * lax.rsqrt(var + 1e-5) * g3_ref[...] + b3_ref[...]
    h3_ref[...] = h3

    z = jnp.dot(h3.astype(jnp.bfloat16), w1_ref[...].astype(jnp.bfloat16),
                preferred_element_type=jnp.float32)
    z = jnp.maximum(z + b1_ref[...], 0.0)
    mu = jnp.mean(z, axis=1, keepdims=True)
    var = jnp.mean((z - mu) ** 2, axis=1, keepdims=True)
    z = (z - mu) * lax.rsqrt(var + 1e-5) * gc_ref[...] + bc_ref[...]
    lg = jnp.dot(z.astype(jnp.bfloat16), w2_ref[...].astype(jnp.bfloat16),
                 preferred_element_type=jnp.float32)
    lg = lg + b2_ref[...]
    m = jnp.max(lg, axis=1, keepdims=True)
    e = jnp.exp(lg - m)
    lg_ref[...] = lg - m - jnp.log(jnp.sum(e, axis=1, keepdims=True))

  return pl.pallas_call(
      body,
      grid=(_N // _BR,),
      in_specs=[
          pl.BlockSpec((_NC, _BR, _D), lambda i: (0, i, 0)),
          pl.BlockSpec((_BR, _D), lambda i: (i, 0)),
          pl.BlockSpec((_D, _D), lambda i: (0, 0)),
          pl.BlockSpec((1, _D), lambda i: (0, 0)),
          pl.BlockSpec((_D, _D), lambda i: (0, 0)),
          pl.BlockSpec((1, _D), lambda i: (0, 0)),
          pl.BlockSpec((1, _D), lambda i: (0, 0)),
          pl.BlockSpec((_D, H2), lambda i: (0, 0)),
          pl.BlockSpec((1, H2), lambda i: (0, 0)),
          pl.BlockSpec((1, H2), lambda i: (0, 0)),
          pl.BlockSpec((1, H2), lambda i: (0, 0)),
          pl.BlockSpec((H2, NCLS), lambda i: (0, 0)),
          pl.BlockSpec((1, NCLS), lambda i: (0, 0)),
      ],
      out_specs=[
          pl.BlockSpec((_BR, NCLS), lambda i: (i, 0)),
          pl.BlockSpec((_BR, _D), lambda i: (i, 0)),
      ],
      out_shape=[
          jax.ShapeDtypeStruct((_N, NCLS), jnp.float32),
          jax.ShapeDtypeStruct((_N, _D), jnp.float32),
      ],
  )(p, h, WrT, br.reshape(1, _D), WoT, g3.reshape(1, _D), b3.reshape(1, _D),
    W1T, b1.reshape(1, H2), gc.reshape(1, H2), bc.reshape(1, H2),
    W2T, b2.reshape(1, NCLS))


def kernel(x, edge_index, ln0_g, ln0_b,
           W_rel1, b_rel1, W_root1, ln1_g, ln1_b,
           W_rel2, b_rel2, W_root2, ln2_g, ln2_b,
           W_rel3, b_rel3, W_root3, ln3_g, ln3_b,
           cls_W1, cls_b1, cls_ln_g, cls_ln_b, cls_W2, cls_b2):
  src = edge_index[0]
  dst = edge_index[1]
  H2 = cls_W1.shape[0]
  NCLS = cls_W2.shape[0]

  xn = _ln_relu(x, ln0_g, ln0_b)
  p1 = _seg_sum_partials(xn, src, dst)
  h1 = _gconv_ln(p1, xn, W_rel1.T, b_rel1, W_root1.T, ln1_g, ln1_b)
  p2 = _seg_sum_partials(h1, src, dst)
  h2 = _gconv_ln(p2, h1, W_rel2.T, b_rel2, W_root2.T, ln2_g, ln2_b)
  p3 = _seg_sum_partials(h2, src, dst)
  logits, h3 = _final_stage(
      p3, h2, W_rel3.T, b_rel3, W_root3.T, ln3_g, ln3_b,
      cls_W1.T, cls_b1, cls_ln_g, cls_ln_b, cls_W2.T, cls_b2, H2, NCLS)
  return (logits, h3)


# speedup vs baseline: 11.1044x; 1.0129x over previous
import functools

import jax
import jax.numpy as jnp
from jax import lax
from jax.experimental import pallas as pl
from jax.experimental.pallas import tpu as pltpu
from jax.experimental.pallas import tpu_sc as plsc

_N = 10000
_E = 320000
_D = 128

_NC = 2
_NS = 16
_NW = _NC * _NS
_EPW = _E // _NW
_K = 80
_NBUF = 3
_NCHUNK = _EPW // _K
_NGROUP = 41
_RPS = 624
_TAIL = _N - _NS * _RPS


def _seg_sum_partials(h, src, dst):
  mesh = plsc.VectorSubcoreMesh(core_axis_name="c", subcore_axis_name="s")

  @functools.partial(
      pl.kernel,
      out_type=jax.ShapeDtypeStruct((_NC, _N, _D), jnp.float32),
      mesh=mesh,
      scratch_types=[
          pltpu.VMEM((_EPW,), jnp.int32),
          pltpu.VMEM((_NBUF, _K), jnp.int32),
          pltpu.VMEM((_NBUF, _K, _D), jnp.float32),
          pltpu.VMEM_SHARED((_N, _D), jnp.float32),
          pltpu.SemaphoreType.DMA,
          pltpu.SemaphoreType.DMA,
          pltpu.SemaphoreType.DMA,
      ],
  )
  def seg_kernel(h_hbm, src_hbm, dst_hbm, out_hbm,
                 srcb, dstb, rows, acc, gsem, ssem, isem):
    c = lax.axis_index("c")
    s = lax.axis_index("s")
    wid = s * _NC + c
    row0 = s * _RPS
    ebase = wid * _EPW

    pltpu.sync_copy(src_hbm.at[pl.ds(ebase, _EPW)], srcb)

    zvec = jnp.zeros((16,), jnp.float32)
    def _zrow(i, carry):
      for b in range(_NBUF):
        for j in range(_D // 16):
          rows[b, i, pl.ds(j * 16, 16)] = zvec
      return carry
    lax.fori_loop(0, _K, _zrow, 0)
    nfull = _RPS // _K
    rem = _RPS - nfull * _K
    for r in range(nfull):
      pltpu.sync_copy(rows.at[r % _NBUF], acc.at[pl.ds(row0 + r * _K, _K)])
    pltpu.sync_copy(rows.at[0, pl.ds(0, rem)],
                    acc.at[pl.ds(row0 + nfull * _K, rem)])
    @pl.when(s == _NS - 1)
    def _zero_tail():
      pltpu.sync_copy(rows.at[1, pl.ds(0, _TAIL)],
                      acc.at[pl.ds(_NS * _RPS, _TAIL)])
    plsc.subcore_barrier()

    def _fire(chunk, b):
      pltpu.async_copy(dst_hbm.at[pl.ds(ebase + chunk * _K, _K)],
                       dstb.at[b], isem)
      pltpu.async_copy(h_hbm.at[srcb.at[pl.ds(chunk * _K, _K)]],
                       rows.at[b], gsem)

    def _drain(b):
      pltpu.make_async_copy(dst_hbm.at[pl.ds(0, _K)], dstb.at[b],
                            isem).wait()
      pltpu.make_async_copy(h_hbm.at[pl.ds(0, _K)], rows.at[b], gsem).wait()

    for b in range(_NBUF):
      _fire(b, b)

    def _group(g, carry):
      sh = []
      for b in range(_NBUF):
        _drain(b)
        sh.append(pltpu.async_copy(rows.at[b], acc.at[dstb.at[b]],
                                   ssem, add=True))
      for b in range(_NBUF):
        sh[b].wait()
        _fire((g + 1) * _NBUF + b, b)
      return carry
    lax.fori_loop(0, _NGROUP - 1, _group, 0)

    sh = []
    for b in range(_NBUF):
      _drain(b)
      sh.append(pltpu.async_copy(rows.at[b], acc.at[dstb.at[b]],
                                 ssem, add=True))
    for b in range(_NBUF):
      sh[b].wait()

    for i, chunk in enumerate(range(_NGROUP * _NBUF, _NCHUNK)):
      _fire(chunk, i)
    sh = []
    for i in range(_NCHUNK - _NGROUP * _NBUF):
      _drain(i)
      sh.append(pltpu.async_copy(rows.at[i], acc.at[dstb.at[i]],
                                 ssem, add=True))
    for h_ in sh:
      h_.wait()

    plsc.subcore_barrier()
    pltpu.sync_copy(acc.at[pl.ds(row0, _RPS)],
                    out_hbm.at[c, pl.ds(row0, _RPS)])
    @pl.when(s == _NS - 1)
    def _out_tail():
      pltpu.sync_copy(acc.at[pl.ds(_NS * _RPS, _TAIL)],
                      out_hbm.at[c, pl.ds(_NS * _RPS, _TAIL)])

  return seg_kernel(h, src, dst)


_BR = 1000


def _ln_relu(x, g, b):
  def body(x_ref, g_ref, b_ref, o_ref):
    xx = x_ref[...]
    mu = jnp.mean(xx, axis=1, keepdims=True)
    var = jnp.mean((xx - mu) ** 2, axis=1, keepdims=True)
    xn = (xx - mu) * lax.rsqrt(var + 1e-5) * g_ref[...] + b_ref[...]
    o_ref[...] = jnp.maximum(xn, 0.0)

  return pl.pallas_call(
      body,
      grid=(_N // _BR,),
      in_specs=[
          pl.BlockSpec((_BR, _D), lambda i: (i, 0)),
          pl.BlockSpec((1, _D), lambda i: (0, 0)),
          pl.BlockSpec((1, _D), lambda i: (0, 0)),
      ],
      out_specs=pl.BlockSpec((_BR, _D), lambda i: (i, 0)),
      out_shape=jax.ShapeDtypeStruct((_N, _D), jnp.float32),
  )(x, g.reshape(1, _D), b.reshape(1, _D))


def _gconv_ln(p, h, WrT, br, WoT, g, b):
  def body(p_ref, h_ref, wr_ref, br_ref, wo_ref, g_ref, b_ref, o_ref):
    agg = p_ref[0] + p_ref[1]
    hh = h_ref[...]
    t = jnp.dot(agg.astype(jnp.bfloat16), wr_ref[...].astype(jnp.bfloat16),
                preferred_element_type=jnp.float32)
    t += jnp.dot(hh.astype(jnp.bfloat16), wo_ref[...].astype(jnp.bfloat16),
                 preferred_element_type=jnp.float32)
    t = hh + jnp.maximum(t + br_ref[...], 0.0)
    mu = jnp.mean(t, axis=1, keepdims=True)
    var = jnp.mean((t - mu) ** 2, axis=1, keepdims=True)
    o_ref[...] = (t - mu) * lax.rsqrt(var + 1e-5) * g_ref[...] + b_ref[...]

  return pl.pallas_call(
      body,
      grid=(_N // _BR,),
      in_specs=[
          pl.BlockSpec((_NC, _BR, _D), lambda i: (0, i, 0)),
          pl.BlockSpec((_BR, _D), lambda i: (i, 0)),
          pl.BlockSpec((_D, _D), lambda i: (0, 0)),
          pl.BlockSpec((1, _D), lambda i: (0, 0)),
          pl.BlockSpec((_D, _D), lambda i: (0, 0)),
          pl.BlockSpec((1, _D), lambda i: (0, 0)),
          pl.BlockSpec((1, _D), lambda i: (0, 0)),
      ],
      out_specs=pl.BlockSpec((_BR, _D), lambda i: (i, 0)),
      out_shape=jax.ShapeDtypeStruct((_N, _D), jnp.float32),
  )(p, h, WrT, br.reshape(1, _D), WoT, g.reshape(1, _D), b.reshape(1, _D))


def _final_stage(p, h, WrT, br, WoT, g3, b3,
                 W1T, b1, gc, bc, W2T, b2, H2, NCLS):
  def body(p_ref, h_ref, wr_ref, br_ref, wo_ref, g3_ref, b3_ref,
           w1_ref, b1_ref, gc_ref, bc_ref, w2_ref, b2_ref,
           lg_ref, h3_ref):
    agg = p_ref[0] + p_ref[1]
    hh = h_ref[...]
    t = jnp.dot(agg.astype(jnp.bfloat16), wr_ref[...].astype(jnp.bfloat16),
                preferred_element_type=jnp.float32)
    t += jnp.dot(hh.astype(jnp.bfloat16), wo_ref[...].astype(jnp.bfloat16),
                 preferred_element_type=jnp.float32)
    t = jnp.maximum(t + br_ref[...], 0.0)
    mu = jnp.mean(t, axis=1, keepdims=True)
    var = jnp.mean((t - mu) ** 2, axis=1, keepdims=True)
    h3 = (t - mu) * lax.rsqrt(var + 1e-5) * g3_ref[...] + b3_ref[...]
    h3_ref[...] = h3

    z = jnp.dot(h3.astype(jnp.bfloat16), w1_ref[...].astype(jnp.bfloat16),
                preferred_element_type=jnp.float32)
    z = jnp.maximum(z + b1_ref[...], 0.0)
    mu = jnp.mean(z, axis=1, keepdims=True)
    var = jnp.mean((z - mu) ** 2, axis=1, keepdims=True)
    z = (z - mu) * lax.rsqrt(var + 1e-5) * gc_ref[...] + bc_ref[...]
    lg = jnp.dot(z.astype(jnp.bfloat16), w2_ref[...].astype(jnp.bfloat16),
                 preferred_element_type=jnp.float32)
    lg = lg + b2_ref[...]
    m = jnp.max(lg, axis=1, keepdims=True)
    e = jnp.exp(lg - m)
    lg_ref[...] = lg - m - jnp.log(jnp.sum(e, axis=1, keepdims=True))

  return pl.pallas_call(
      body,
      grid=(_N // _BR,),
      in_specs=[
          pl.BlockSpec((_NC, _BR, _D), lambda i: (0, i, 0)),
          pl.BlockSpec((_BR, _D), lambda i: (i, 0)),
          pl.BlockSpec((_D, _D), lambda i: (0, 0)),
          pl.BlockSpec((1, _D), lambda i: (0, 0)),
          pl.BlockSpec((_D, _D), lambda i: (0, 0)),
          pl.BlockSpec((1, _D), lambda i: (0, 0)),
          pl.BlockSpec((1, _D), lambda i: (0, 0)),
          pl.BlockSpec((_D, H2), lambda i: (0, 0)),
          pl.BlockSpec((1, H2), lambda i: (0, 0)),
          pl.BlockSpec((1, H2), lambda i: (0, 0)),
          pl.BlockSpec((1, H2), lambda i: (0, 0)),
          pl.BlockSpec((H2, NCLS), lambda i: (0, 0)),
          pl.BlockSpec((1, NCLS), lambda i: (0, 0)),
      ],
      out_specs=[
          pl.BlockSpec((_BR, NCLS), lambda i: (i, 0)),
          pl.BlockSpec((_BR, _D), lambda i: (i, 0)),
      ],
      out_shape=[
          jax.ShapeDtypeStruct((_N, NCLS), jnp.float32),
          jax.ShapeDtypeStruct((_N, _D), jnp.float32),
      ],
  )(p, h, WrT, br.reshape(1, _D), WoT, g3.reshape(1, _D), b3.reshape(1, _D),
    W1T, b1.reshape(1, H2), gc.reshape(1, H2), bc.reshape(1, H2),
    W2T, b2.reshape(1, NCLS))


def kernel(x, edge_index, ln0_g, ln0_b,
           W_rel1, b_rel1, W_root1, ln1_g, ln1_b,
           W_rel2, b_rel2, W_root2, ln2_g, ln2_b,
           W_rel3, b_rel3, W_root3, ln3_g, ln3_b,
           cls_W1, cls_b1, cls_ln_g, cls_ln_b, cls_W2, cls_b2):
  src = edge_index[0]
  dst = edge_index[1]
  H2 = cls_W1.shape[0]
  NCLS = cls_W2.shape[0]

  xn = _ln_relu(x, ln0_g, ln0_b)
  p1 = _seg_sum_partials(xn, src, dst)
  h1 = _gconv_ln(p1, xn, W_rel1.T, b_rel1, W_root1.T, ln1_g, ln1_b)
  p2 = _seg_sum_partials(h1, src, dst)
  h2 = _gconv_ln(p2, h1, W_rel2.T, b_rel2, W_root2.T, ln2_g, ln2_b)
  p3 = _seg_sum_partials(h2, src, dst)
  logits, h3 = _final_stage(
      p3, h2, W_rel3.T, b_rel3, W_root3.T, ln3_g, ln3_b,
      cls_W1.T, cls_b1, cls_ln_g, cls_ln_b, cls_W2.T, cls_b2, H2, NCLS)
  return (logits, h3)


# speedup vs baseline: 11.4248x; 1.0289x over previous
import functools

import jax
import jax.numpy as jnp
from jax import lax
from jax.experimental import pallas as pl
from jax.experimental.pallas import tpu as pltpu
from jax.experimental.pallas import tpu_sc as plsc

_N = 10000
_E = 320000
_D = 128

_NC = 2
_NS = 16
_NW = _NC * _NS
_EPW = _E // _NW
_K = 80
_NBUF = 3
_NCHUNK = _EPW // _K
_NGROUP = 41
_RPS = 624
_TAIL = _N - _NS * _RPS


def _seg_sum_partials(h, src, dst):
  mesh = plsc.VectorSubcoreMesh(core_axis_name="c", subcore_axis_name="s")

  @functools.partial(
      pl.kernel,
      out_type=jax.ShapeDtypeStruct((_NC, _N, _D), jnp.float32),
      mesh=mesh,
      scratch_types=[
          pltpu.VMEM((_EPW,), jnp.int32),
          pltpu.VMEM((_NBUF, _K), jnp.int32),
          pltpu.VMEM((_NBUF, _K, _D), jnp.float32),
          pltpu.VMEM_SHARED((_N, _D), jnp.float32),
          pltpu.SemaphoreType.DMA,
          pltpu.SemaphoreType.DMA,
          pltpu.SemaphoreType.DMA,
      ],
  )
  def seg_kernel(h_hbm, src_hbm, dst_hbm, out_hbm,
                 srcb, dstb, rows, acc, gsem, ssem, isem):
    c = lax.axis_index("c")
    s = lax.axis_index("s")
    wid = s * _NC + c
    row0 = s * _RPS
    ebase = wid * _EPW

    pltpu.sync_copy(src_hbm.at[pl.ds(ebase, _EPW)], srcb)

    zvec = jnp.zeros((16,), jnp.float32)
    def _zrow(i, carry):
      for b in range(_NBUF):
        for j in range(_D // 16):
          rows[b, i, pl.ds(j * 16, 16)] = zvec
      return carry
    lax.fori_loop(0, _K, _zrow, 0)
    nfull = _RPS // _K
    rem = _RPS - nfull * _K
    for r in range(nfull):
      pltpu.sync_copy(rows.at[r % _NBUF], acc.at[pl.ds(row0 + r * _K, _K)])
    pltpu.sync_copy(rows.at[0, pl.ds(0, rem)],
                    acc.at[pl.ds(row0 + nfull * _K, rem)])
    @pl.when(s == _NS - 1)
    def _zero_tail():
      pltpu.sync_copy(rows.at[1, pl.ds(0, _TAIL)],
                      acc.at[pl.ds(_NS * _RPS, _TAIL)])
    plsc.subcore_barrier()

    def _fire(chunk, b):
      pltpu.async_copy(dst_hbm.at[pl.ds(ebase + chunk * _K, _K)],
                       dstb.at[b], isem)
      pltpu.async_copy(h_hbm.at[srcb.at[pl.ds(chunk * _K, _K)]],
                       rows.at[b], gsem)

    def _drain(b):
      pltpu.make_async_copy(dst_hbm.at[pl.ds(0, _K)], dstb.at[b],
                            isem).wait()
      pltpu.make_async_copy(h_hbm.at[pl.ds(0, _K)], rows.at[b], gsem).wait()

    for b in range(_NBUF):
      _fire(b, b)

    def _group(g, carry):
      sh = []
      for b in range(_NBUF):
        _drain(b)
        sh.append(pltpu.async_copy(rows.at[b], acc.at[dstb.at[b]],
                                   ssem, add=True))
      for b in range(_NBUF):
        sh[b].wait()
        _fire((g + 1) * _NBUF + b, b)
      return carry
    lax.fori_loop(0, _NGROUP - 1, _group, 0)

    sh = []
    for b in range(_NBUF):
      _drain(b)
      sh.append(pltpu.async_copy(rows.at[b], acc.at[dstb.at[b]],
                                 ssem, add=True))
    for b in range(_NBUF):
      sh[b].wait()

    for i, chunk in enumerate(range(_NGROUP * _NBUF, _NCHUNK)):
      _fire(chunk, i)
    sh = []
    for i in range(_NCHUNK - _NGROUP * _NBUF):
      _drain(i)
      sh.append(pltpu.async_copy(rows.at[i], acc.at[dstb.at[i]],
                                 ssem, add=True))
    for h_ in sh:
      h_.wait()

    plsc.subcore_barrier()
    pltpu.sync_copy(acc.at[pl.ds(row0, _RPS)],
                    out_hbm.at[c, pl.ds(row0, _RPS)])
    @pl.when(s == _NS - 1)
    def _out_tail():
      pltpu.sync_copy(acc.at[pl.ds(_NS * _RPS, _TAIL)],
                      out_hbm.at[c, pl.ds(_NS * _RPS, _TAIL)])

  return seg_kernel(h, src, dst)


_BR = 2000


def _ln_relu(x, g, b):
  def body(x_ref, g_ref, b_ref, o_ref):
    xx = x_ref[...]
    mu = jnp.mean(xx, axis=1, keepdims=True)
    var = jnp.mean((xx - mu) ** 2, axis=1, keepdims=True)
    xn = (xx - mu) * lax.rsqrt(var + 1e-5) * g_ref[...] + b_ref[...]
    o_ref[...] = jnp.maximum(xn, 0.0)

  return pl.pallas_call(
      body,
      grid=(_N // _BR,),
      in_specs=[
          pl.BlockSpec((_BR, _D), lambda i: (i, 0)),
          pl.BlockSpec((1, _D), lambda i: (0, 0)),
          pl.BlockSpec((1, _D), lambda i: (0, 0)),
      ],
      out_specs=pl.BlockSpec((_BR, _D), lambda i: (i, 0)),
      out_shape=jax.ShapeDtypeStruct((_N, _D), jnp.float32),
  )(x, g.reshape(1, _D), b.reshape(1, _D))


def _gconv_ln(p, h, WrT, br, WoT, g, b):
  def body(p_ref, h_ref, wr_ref, br_ref, wo_ref, g_ref, b_ref, o_ref):
    agg = p_ref[0] + p_ref[1]
    hh = h_ref[...]
    t = jnp.dot(agg.astype(jnp.bfloat16), wr_ref[...].astype(jnp.bfloat16),
                preferred_element_type=jnp.float32)
    t += jnp.dot(hh.astype(jnp.bfloat16), wo_ref[...].astype(jnp.bfloat16),
                 preferred_element_type=jnp.float32)
    t = hh + jnp.maximum(t + br_ref[...], 0.0)
    mu = jnp.mean(t, axis=1, keepdims=True)
    var = jnp.mean((t - mu) ** 2, axis=1, keepdims=True)
    o_ref[...] = (t - mu) * lax.rsqrt(var + 1e-5) * g_ref[...] + b_ref[...]

  return pl.pallas_call(
      body,
      grid=(_N // _BR,),
      in_specs=[
          pl.BlockSpec((_NC, _BR, _D), lambda i: (0, i, 0)),
          pl.BlockSpec((_BR, _D), lambda i: (i, 0)),
          pl.BlockSpec((_D, _D), lambda i: (0, 0)),
          pl.BlockSpec((1, _D), lambda i: (0, 0)),
          pl.BlockSpec((_D, _D), lambda i: (0, 0)),
          pl.BlockSpec((1, _D), lambda i: (0, 0)),
          pl.BlockSpec((1, _D), lambda i: (0, 0)),
      ],
      out_specs=pl.BlockSpec((_BR, _D), lambda i: (i, 0)),
      out_shape=jax.ShapeDtypeStruct((_N, _D), jnp.float32),
  )(p, h, WrT, br.reshape(1, _D), WoT, g.reshape(1, _D), b.reshape(1, _D))


def _final_stage(p, h, WrT, br, WoT, g3, b3,
                 W1T, b1, gc, bc, W2T, b2, H2, NCLS):
  def body(p_ref, h_ref, wr_ref, br_ref, wo_ref, g3_ref, b3_ref,
           w1_ref, b1_ref, gc_ref, bc_ref, w2_ref, b2_ref,
           lg_ref, h3_ref):
    agg = p_ref[0] + p_ref[1]
    hh = h_ref[...]
    t = jnp.dot(agg.astype(jnp.bfloat16), wr_ref[...].astype(jnp.bfloat16),
                preferred_element_type=jnp.float32)
    t += jnp.dot(hh.astype(jnp.bfloat16), wo_ref[...].astype(jnp.bfloat16),
                 preferred_element_type=jnp.float32)
    t = jnp.maximum(t + br_ref[...], 0.0)
    mu = jnp.mean(t, axis=1, keepdims=True)
    var = jnp.mean((t - mu) ** 2, axis=1, keepdims=True)
    h3 = (t - mu) * lax.rsqrt(var + 1e-5) * g3_ref[...] + b3_ref[...]
    h3_ref[...] = h3

    z = jnp.dot(h3.astype(jnp.bfloat16), w1_ref[...].astype(jnp.bfloat16),
                preferred_element_type=jnp.float32)
    z = jnp.maximum(z + b1_ref[...], 0.0)
    mu = jnp.mean(z, axis=1, keepdims=True)
    var = jnp.mean((z - mu) ** 2, axis=1, keepdims=True)
    z = (z - mu) * lax.rsqrt(var + 1e-5) * gc_ref[...] + bc_ref[...]
    lg = jnp.dot(z.astype(jnp.bfloat16), w2_ref[...].astype(jnp.bfloat16),
                 preferred_element_type=jnp.float32)
    lg = lg + b2_ref[...]
    m = jnp.max(lg, axis=1, keepdims=True)
    e = jnp.exp(lg - m)
    lg_ref[...] = lg - m - jnp.log(jnp.sum(e, axis=1, keepdims=True))

  return pl.pallas_call(
      body,
      grid=(_N // _BR,),
      in_specs=[
          pl.BlockSpec((_NC, _BR, _D), lambda i: (0, i, 0)),
          pl.BlockSpec((_BR, _D), lambda i: (i, 0)),
          pl.BlockSpec((_D, _D), lambda i: (0, 0)),
          pl.BlockSpec((1, _D), lambda i: (0, 0)),
          pl.BlockSpec((_D, _D), lambda i: (0, 0)),
          pl.BlockSpec((1, _D), lambda i: (0, 0)),
          pl.BlockSpec((1, _D), lambda i: (0, 0)),
          pl.BlockSpec((_D, H2), lambda i: (0, 0)),
          pl.BlockSpec((1, H2), lambda i: (0, 0)),
          pl.BlockSpec((1, H2), lambda i: (0, 0)),
          pl.BlockSpec((1, H2), lambda i: (0, 0)),
          pl.BlockSpec((H2, NCLS), lambda i: (0, 0)),
          pl.BlockSpec((1, NCLS), lambda i: (0, 0)),
      ],
      out_specs=[
          pl.BlockSpec((_BR, NCLS), lambda i: (i, 0)),
          pl.BlockSpec((_BR, _D), lambda i: (i, 0)),
      ],
      out_shape=[
          jax.ShapeDtypeStruct((_N, NCLS), jnp.float32),
          jax.ShapeDtypeStruct((_N, _D), jnp.float32),
      ],
  )(p, h, WrT, br.reshape(1, _D), WoT, g3.reshape(1, _D), b3.reshape(1, _D),
    W1T, b1.reshape(1, H2), gc.reshape(1, H2), bc.reshape(1, H2),
    W2T, b2.reshape(1, NCLS))


def kernel(x, edge_index, ln0_g, ln0_b,
           W_rel1, b_rel1, W_root1, ln1_g, ln1_b,
           W_rel2, b_rel2, W_root2, ln2_g, ln2_b,
           W_rel3, b_rel3, W_root3, ln3_g, ln3_b,
           cls_W1, cls_b1, cls_ln_g, cls_ln_b, cls_W2, cls_b2):
  src = edge_index[0]
  dst = edge_index[1]
  H2 = cls_W1.shape[0]
  NCLS = cls_W2.shape[0]

  xn = _ln_relu(x, ln0_g, ln0_b)
  p1 = _seg_sum_partials(xn, src, dst)
  h1 = _gconv_ln(p1, xn, W_rel1.T, b_rel1, W_root1.T, ln1_g, ln1_b)
  p2 = _seg_sum_partials(h1, src, dst)
  h2 = _gconv_ln(p2, h1, W_rel2.T, b_rel2, W_root2.T, ln2_g, ln2_b)
  p3 = _seg_sum_partials(h2, src, dst)
  logits, h3 = _final_stage(
      p3, h2, W_rel3.T, b_rel3, W_root3.T, ln3_g, ln3_b,
      cls_W1.T, cls_b1, cls_ln_g, cls_ln_b, cls_W2.T, cls_b2, H2, NCLS)
  return (logits, h3)


# speedup vs baseline: 11.5597x; 1.0118x over previous
import functools

import jax
import jax.numpy as jnp
from jax import lax
from jax.experimental import pallas as pl
from jax.experimental.pallas import tpu as pltpu
from jax.experimental.pallas import tpu_sc as plsc

_N = 10000
_E = 320000
_D = 128

_NC = 2
_NS = 16
_NW = _NC * _NS
_EPW = _E // _NW
_K = 80
_NBUF = 3
_NCHUNK = _EPW // _K
_NGROUP = 41
_RPS = 624
_TAIL = _N - _NS * _RPS


def _seg_sum_partials(h, src, dst):
  mesh = plsc.VectorSubcoreMesh(core_axis_name="c", subcore_axis_name="s")

  @functools.partial(
      pl.kernel,
      out_type=jax.ShapeDtypeStruct((_NC, _N, _D), jnp.float32),
      mesh=mesh,
      scratch_types=[
          pltpu.VMEM((_EPW,), jnp.int32),
          pltpu.VMEM((_NBUF, _K), jnp.int32),
          pltpu.VMEM((_NBUF, _K, _D), jnp.float32),
          pltpu.VMEM_SHARED((_N, _D), jnp.float32),
          pltpu.SemaphoreType.DMA,
          pltpu.SemaphoreType.DMA,
          pltpu.SemaphoreType.DMA,
      ],
  )
  def seg_kernel(h_hbm, src_hbm, dst_hbm, out_hbm,
                 srcb, dstb, rows, acc, gsem, ssem, isem):
    c = lax.axis_index("c")
    s = lax.axis_index("s")
    wid = s * _NC + c
    row0 = s * _RPS
    ebase = wid * _EPW

    pltpu.sync_copy(src_hbm.at[pl.ds(ebase, _EPW)], srcb)

    zvec = jnp.zeros((16,), jnp.float32)
    def _zrow(i, carry):
      for b in range(_NBUF):
        for j in range(_D // 16):
          rows[b, i, pl.ds(j * 16, 16)] = zvec
      return carry
    lax.fori_loop(0, _K, _zrow, 0)
    nfull = _RPS // _K
    rem = _RPS - nfull * _K
    for r in range(nfull):
      pltpu.sync_copy(rows.at[r % _NBUF], acc.at[pl.ds(row0 + r * _K, _K)])
    pltpu.sync_copy(rows.at[0, pl.ds(0, rem)],
                    acc.at[pl.ds(row0 + nfull * _K, rem)])
    @pl.when(s == _NS - 1)
    def _zero_tail():
      pltpu.sync_copy(rows.at[1, pl.ds(0, _TAIL)],
                      acc.at[pl.ds(_NS * _RPS, _TAIL)])
    plsc.subcore_barrier()

    def _fire(chunk, b):
      pltpu.async_copy(dst_hbm.at[pl.ds(ebase + chunk * _K, _K)],
                       dstb.at[b], isem)
      pltpu.async_copy(h_hbm.at[srcb.at[pl.ds(chunk * _K, _K)]],
                       rows.at[b], gsem)

    def _drain(b):
      pltpu.make_async_copy(dst_hbm.at[pl.ds(0, _K)], dstb.at[b],
                            isem).wait()
      pltpu.make_async_copy(h_hbm.at[pl.ds(0, _K)], rows.at[b], gsem).wait()

    for b in range(_NBUF):
      _fire(b, b)

    def _group(g, carry):
      sh = []
      for b in range(_NBUF):
        _drain(b)
        sh.append(pltpu.async_copy(rows.at[b], acc.at[dstb.at[b]],
                                   ssem, add=True))
      for b in range(_NBUF):
        sh[b].wait()
        _fire((g + 1) * _NBUF + b, b)
      return carry
    lax.fori_loop(0, _NGROUP - 1, _group, 0)

    sh = []
    for b in range(_NBUF):
      _drain(b)
      sh.append(pltpu.async_copy(rows.at[b], acc.at[dstb.at[b]],
                                 ssem, add=True))
    for b in range(_NBUF):
      sh[b].wait()

    for i, chunk in enumerate(range(_NGROUP * _NBUF, _NCHUNK)):
      _fire(chunk, i)
    sh = []
    for i in range(_NCHUNK - _NGROUP * _NBUF):
      _drain(i)
      sh.append(pltpu.async_copy(rows.at[i], acc.at[dstb.at[i]],
                                 ssem, add=True))
    for h_ in sh:
      h_.wait()

    plsc.subcore_barrier()
    pltpu.sync_copy(acc.at[pl.ds(row0, _RPS)],
                    out_hbm.at[c, pl.ds(row0, _RPS)])
    @pl.when(s == _NS - 1)
    def _out_tail():
      pltpu.sync_copy(acc.at[pl.ds(_NS * _RPS, _TAIL)],
                      out_hbm.at[c, pl.ds(_NS * _RPS, _TAIL)])

  return seg_kernel(h, src, dst)


_BR = 5000


def _ln_relu(x, g, b):
  def body(x_ref, g_ref, b_ref, o_ref):
    xx = x_ref[...]
    mu = jnp.mean(xx, axis=1, keepdims=True)
    var = jnp.mean((xx - mu) ** 2, axis=1, keepdims=True)
    xn = (xx - mu) * lax.rsqrt(var + 1e-5) * g_ref[...] + b_ref[...]
    o_ref[...] = jnp.maximum(xn, 0.0)

  return pl.pallas_call(
      body,
      grid=(_N // _BR,),
      in_specs=[
          pl.BlockSpec((_BR, _D), lambda i: (i, 0)),
          pl.BlockSpec((1, _D), lambda i: (0, 0)),
          pl.BlockSpec((1, _D), lambda i: (0, 0)),
      ],
      out_specs=pl.BlockSpec((_BR, _D), lambda i: (i, 0)),
      out_shape=jax.ShapeDtypeStruct((_N, _D), jnp.float32),
  )(x, g.reshape(1, _D), b.reshape(1, _D))


def _gconv_ln(p, h, WrT, br, WoT, g, b):
  def body(p_ref, h_ref, wr_ref, br_ref, wo_ref, g_ref, b_ref, o_ref):
    agg = p_ref[0] + p_ref[1]
    hh = h_ref[...]
    t = jnp.dot(agg.astype(jnp.bfloat16), wr_ref[...].astype(jnp.bfloat16),
                preferred_element_type=jnp.float32)
    t += jnp.dot(hh.astype(jnp.bfloat16), wo_ref[...].astype(jnp.bfloat16),
                 preferred_element_type=jnp.float32)
    t = hh + jnp.maximum(t + br_ref[...], 0.0)
    mu = jnp.mean(t, axis=1, keepdims=True)
    var = jnp.mean((t - mu) ** 2, axis=1, keepdims=True)
    o_ref[...] = (t - mu) * lax.rsqrt(var + 1e-5) * g_ref[...] + b_ref[...]

  return pl.pallas_call(
      body,
      grid=(_N // _BR,),
      in_specs=[
          pl.BlockSpec((_NC, _BR, _D), lambda i: (0, i, 0)),
          pl.BlockSpec((_BR, _D), lambda i: (i, 0)),
          pl.BlockSpec((_D, _D), lambda i: (0, 0)),
          pl.BlockSpec((1, _D), lambda i: (0, 0)),
          pl.BlockSpec((_D, _D), lambda i: (0, 0)),
          pl.BlockSpec((1, _D), lambda i: (0, 0)),
          pl.BlockSpec((1, _D), lambda i: (0, 0)),
      ],
      out_specs=pl.BlockSpec((_BR, _D), lambda i: (i, 0)),
      out_shape=jax.ShapeDtypeStruct((_N, _D), jnp.float32),
  )(p, h, WrT, br.reshape(1, _D), WoT, g.reshape(1, _D), b.reshape(1, _D))


def _final_stage(p, h, WrT, br, WoT, g3, b3,
                 W1T, b1, gc, bc, W2T, b2, H2, NCLS):
  def body(p_ref, h_ref, wr_ref, br_ref, wo_ref, g3_ref, b3_ref,
           w1_ref, b1_ref, gc_ref, bc_ref, w2_ref, b2_ref,
           lg_ref, h3_ref):
    agg = p_ref[0] + p_ref[1]
    hh = h_ref[...]
    t = jnp.dot(agg.astype(jnp.bfloat16), wr_ref[...].astype(jnp.bfloat16),
                preferred_element_type=jnp.float32)
    t += jnp.dot(hh.astype(jnp.bfloat16), wo_ref[...].astype(jnp.bfloat16),
                 preferred_element_type=jnp.float32)
    t = jnp.maximum(t + br_ref[...], 0.0)
    mu = jnp.mean(t, axis=1, keepdims=True)
    var = jnp.mean((t - mu) ** 2, axis=1, keepdims=True)
    h3 = (t - mu) * lax.rsqrt(var + 1e-5) * g3_ref[...] + b3_ref[...]
    h3_ref[...] = h3

    z = jnp.dot(h3.astype(jnp.bfloat16), w1_ref[...].astype(jnp.bfloat16),
                preferred_element_type=jnp.float32)
    z = jnp.maximum(z + b1_ref[...], 0.0)
    mu = jnp.mean(z, axis=1, keepdims=True)
    var = jnp.mean((z - mu) ** 2, axis=1, keepdims=True)
    z = (z - mu) * lax.rsqrt(var + 1e-5) * gc_ref[...] + bc_ref[...]
    lg = jnp.dot(z.astype(jnp.bfloat16), w2_ref[...].astype(jnp.bfloat16),
                 preferred_element_type=jnp.float32)
    lg = lg + b2_ref[...]
    m = jnp.max(lg, axis=1, keepdims=True)
    e = jnp.exp(lg - m)
    lg_ref[...] = lg - m - jnp.log(jnp.sum(e, axis=1, keepdims=True))

  return pl.pallas_call(
      body,
      grid=(_N // _BR,),
      in_specs=[
          pl.BlockSpec((_NC, _BR, _D), lambda i: (0, i, 0)),
          pl.BlockSpec((_BR, _D), lambda i: (i, 0)),
          pl.BlockSpec((_D, _D), lambda i: (0, 0)),
          pl.BlockSpec((1, _D), lambda i: (0, 0)),
          pl.BlockSpec((_D, _D), lambda i: (0, 0)),
          pl.BlockSpec((1, _D), lambda i: (0, 0)),
          pl.BlockSpec((1, _D), lambda i: (0, 0)),
          pl.BlockSpec((_D, H2), lambda i: (0, 0)),
          pl.BlockSpec((1, H2), lambda i: (0, 0)),
          pl.BlockSpec((1, H2), lambda i: (0, 0)),
          pl.BlockSpec((1, H2), lambda i: (0, 0)),
          pl.BlockSpec((H2, NCLS), lambda i: (0, 0)),
          pl.BlockSpec((1, NCLS), lambda i: (0, 0)),
      ],
      out_specs=[
          pl.BlockSpec((_BR, NCLS), lambda i: (i, 0)),
          pl.BlockSpec((_BR, _D), lambda i: (i, 0)),
      ],
      out_shape=[
          jax.ShapeDtypeStruct((_N, NCLS), jnp.float32),
          jax.ShapeDtypeStruct((_N, _D), jnp.float32),
      ],
  )(p, h, WrT, br.reshape(1, _D), WoT, g3.reshape(1, _D), b3.reshape(1, _D),
    W1T, b1.reshape(1, H2), gc.reshape(1, H2), bc.reshape(1, H2),
    W2T, b2.reshape(1, NCLS))


def kernel(x, edge_index, ln0_g, ln0_b,
           W_rel1, b_rel1, W_root1, ln1_g, ln1_b,
           W_rel2, b_rel2, W_root2, ln2_g, ln2_b,
           W_rel3, b_rel3, W_root3, ln3_g, ln3_b,
           cls_W1, cls_b1, cls_ln_g, cls_ln_b, cls_W2, cls_b2):
  src = edge_index[0]
  dst = edge_index[1]
  H2 = cls_W1.shape[0]
  NCLS = cls_W2.shape[0]

  xn = _ln_relu(x, ln0_g, ln0_b)
  p1 = _seg_sum_partials(xn, src, dst)
  h1 = _gconv_ln(p1, xn, W_rel1.T, b_rel1, W_root1.T, ln1_g, ln1_b)
  p2 = _seg_sum_partials(h1, src, dst)
  h2 = _gconv_ln(p2, h1, W_rel2.T, b_rel2, W_root2.T, ln2_g, ln2_b)
  p3 = _seg_sum_partials(h2, src, dst)
  logits, h3 = _final_stage(
      p3, h2, W_rel3.T, b_rel3, W_root3.T, ln3_g, ln3_b,
      cls_W1.T, cls_b1, cls_ln_g, cls_ln_b, cls_W2.T, cls_b2, H2, NCLS)
  return (logits, h3)
